# baseline (speedup 1.0000x reference)
import functools

import jax
import jax.numpy as jnp
from jax import lax
from jax.experimental import pallas as pl
from jax.experimental.pallas import tpu as pltpu
from jax.experimental.pallas import tpu_sc as plsc

_NC = 2
_NS = 16
_NW = _NC * _NS


def _sc_gather_pairs(up, bp, user_id, isbn_id):
    B = user_id.shape[0]
    W = up.shape[1]
    bpw = B // _NW
    CHR = 256
    mesh = plsc.VectorSubcoreMesh(core_axis_name="c", subcore_axis_name="s")
    out_ty = jax.ShapeDtypeStruct((B, W), up.dtype)
    L = 16

    @functools.partial(
        pl.kernel,
        mesh=mesh,
        out_type=(out_ty, out_ty),
        scratch_types=[
            pltpu.VMEM((bpw,), jnp.int32),
            pltpu.VMEM((CHR, W), jnp.float32),
        ],
    )
    def k(ut_hbm, bt_hbm, uid_hbm, bid_hbm, uout_hbm, bout_hbm,
          idx_v, rows_v):
        wid = lax.axis_index("s") * _NC + lax.axis_index("c")
        base = wid * bpw

        def gather_to(table_hbm, id_hbm, out_hbm):
            pltpu.sync_copy(id_hbm.at[pl.ds(base, bpw)], idx_v)

            @pl.loop(0, bpw, step=L)
            def _(j):
                idx_v[pl.ds(j, L)] = lax.div(idx_v[pl.ds(j, L)], 2)

            @pl.loop(0, bpw, step=CHR)
            def _(c):
                pltpu.sync_copy(table_hbm.at[idx_v.at[pl.ds(c, CHR)]], rows_v)
                pltpu.sync_copy(rows_v, out_hbm.at[pl.ds(base + c, CHR)])

        gather_to(ut_hbm, uid_hbm, uout_hbm)
        gather_to(bt_hbm, bid_hbm, bout_hbm)

    return k(up, bp, user_id, isbn_id)


def _mlp_body(u_ref, b_ref, pu_ref, pb_ref, w1a_ref, w1b_ref, b1_ref,
              w2_ref, b2_ref, w3t_ref, b3_ref, o_ref):
    d = w1a_ref.shape[0]
    upair = u_ref[...]
    bpair = b_ref[...]
    u = jnp.where(pu_ref[...] > 0.5, upair[:, d:], upair[:, :d])
    b = jnp.where(pb_ref[...] > 0.5, bpair[:, d:], bpair[:, :d])
    h = (
        jnp.dot(u, w1a_ref[...], preferred_element_type=jnp.float32)
        + jnp.dot(b, w1b_ref[...], preferred_element_type=jnp.float32)
        + b1_ref[...]
    )
    h = jnp.maximum(h, 0.0)
    h = jnp.dot(h, w2_ref[...], preferred_element_type=jnp.float32) + b2_ref[...]
    h = jnp.maximum(h, 0.0)
    o_ref[...] = (
        jnp.sum(h * w3t_ref[...], axis=1, keepdims=True) + b3_ref[...]
    )


def _tc_mlp(u_pack, b_pack, pu, pb, W1, b1, W2, b2, W3, b3, block_b=2048):
    B = u_pack.shape[0]
    D = W1.shape[0] // 2
    H1 = W1.shape[1]
    H2 = W2.shape[1]
    w1a = W1[:D]
    w1b = W1[D:]
    b1r = b1.reshape(1, H1)
    b2r = b2.reshape(1, H2)
    w3t = W3.reshape(1, H2)
    b3r = b3.reshape(1, 1)
    grid = (B // block_b,)

    def full(shape):
        return pl.BlockSpec(shape, lambda i: (0, 0))

    return pl.pallas_call(
        _mlp_body,
        grid=grid,
        in_specs=[
            pl.BlockSpec((block_b, 2 * D), lambda i: (i, 0)),
            pl.BlockSpec((block_b, 2 * D), lambda i: (i, 0)),
            pl.BlockSpec((block_b, 1), lambda i: (i, 0)),
            pl.BlockSpec((block_b, 1), lambda i: (i, 0)),
            full((D, H1)),
            full((D, H1)),
            full((1, H1)),
            full((H1, H2)),
            full((1, H2)),
            full((1, H2)),
            full((1, 1)),
        ],
        out_specs=pl.BlockSpec((block_b, 1), lambda i: (i, 0)),
        out_shape=jax.ShapeDtypeStruct((B, 1), jnp.float32),
    )(u_pack, b_pack, pu, pb, w1a, w1b, b1r, W2, b2r, w3t, b3r)


def kernel(user_id, isbn_id, user_table, book_table, W1, b1, W2, b2, W3, b3):
    D = user_table.shape[1]
    uh = user_table.shape[0] // 2
    bh = book_table.shape[0] // 2
    up = user_table[: 2 * uh].reshape(uh, 2 * D)
    bp = book_table[: 2 * bh].reshape(bh, 2 * D)
    uid = user_id.astype(jnp.int32)
    bid = isbn_id.astype(jnp.int32)
    u_pack, b_pack = _sc_gather_pairs(up, bp, uid, bid)
    B = uid.shape[0]
    pu = (uid % 2).astype(jnp.float32).reshape(B, 1)
    pb = (bid % 2).astype(jnp.float32).reshape(B, 1)
    return _tc_mlp(u_pack, b_pack, pu, pb, W1, b1, W2, b2, W3, b3)
